# baseline (speedup 1.0000x reference)
import functools

import jax
import jax.numpy as jnp
from jax import lax
from jax.experimental import pallas as pl
from jax.experimental.pallas import tpu as pltpu
from jax.experimental.pallas import tpu_sc as plsc

NN = 10000
EE = 320000
SLOT_N = 32
NSLOT = 320
NPAD = NSLOT * SLOT_N
NTILE = 32
PASSES = NSLOT // NTILE
EBLEN = 328
BS = 32
NB = EE // BS
BM = 400


def _att_epilogue(h, asv, adv, as_ref, ad_ref, ws_ref, H):
    bm = h.shape[0]
    ch = asv.shape[1]
    h3 = h.reshape(bm, H, ch)
    a_s = (h3 * asv[None, :, :]).sum(-1)
    a_d = (h3 * adv[None, :, :]).sum(-1)
    pad = jnp.zeros((bm, 128 - H), jnp.float32)
    as_ref[...] = jnp.concatenate([a_s, pad], axis=1)
    ad_ref[...] = jnp.concatenate([a_d, pad], axis=1)
    al = a_s + a_d
    al = jnp.where(al > 0, al, 0.2 * al)
    ws_ref[...] = jnp.exp(al)


def _mm_att1_body(x_ref, w_ref, asv_ref, adv_ref, h_ref, as_ref, ad_ref,
                  ws_ref, *, H):
    h = jnp.dot(x_ref[...], w_ref[...], preferred_element_type=jnp.float32)
    h_ref[...] = h
    _att_epilogue(h, asv_ref[...], adv_ref[...], as_ref, ad_ref, ws_ref, H)


def _mm_att_body(acc_ref, den_ref, wsp_ref, hp_ref, b_ref, w_ref, asv_ref,
                 adv_ref, h_ref, as_ref, ad_ref, ws_ref, *, Hp, H):
    bm = acc_ref.shape[0]
    din = acc_ref.shape[1]
    chp = din // Hp
    acc = acc_ref[...].reshape(bm, Hp, chp)
    hp = hp_ref[...].reshape(bm, Hp, chp)
    wsp = wsp_ref[...][:, :, None]
    den = den_ref[...][:, :Hp][:, :, None]
    xn = (acc + wsp * hp) / (den + wsp + 1e-16)
    xn = xn.reshape(bm, din) + b_ref[...]
    xn = jnp.where(xn > 0, xn, jnp.exp(xn) - 1.0)
    h = jnp.dot(xn, w_ref[...], preferred_element_type=jnp.float32)
    h_ref[...] = h
    _att_epilogue(h, asv_ref[...], adv_ref[...], as_ref, ad_ref, ws_ref, H)


def _final_body(acc_ref, den_ref, wsp_ref, hp_ref, b_ref, o_ref):
    Hp, chp = 6, 128
    bm = acc_ref.shape[0]
    acc = acc_ref[...].reshape(bm, Hp, chp)
    hp = hp_ref[...].reshape(bm, Hp, chp)
    wsp = wsp_ref[...][:, :, None]
    den = den_ref[...][:, :Hp][:, :, None]
    y = (acc + wsp * hp) / (den + wsp + 1e-16)
    y = y + b_ref[...].reshape(1, Hp, chp)
    o_ref[...] = y.mean(axis=1)[:, :121]


def _mm_att1(x, W, asv, adv, H):
    m, k = x.shape
    n = W.shape[1]
    ch = asv.shape[1]
    return pl.pallas_call(
        functools.partial(_mm_att1_body, H=H),
        grid=(m // BM,),
        in_specs=[
            pl.BlockSpec((BM, k), lambda i: (i, 0)),
            pl.BlockSpec((k, n), lambda i: (0, 0)),
            pl.BlockSpec((H, ch), lambda i: (0, 0)),
            pl.BlockSpec((H, ch), lambda i: (0, 0)),
        ],
        out_specs=[
            pl.BlockSpec((BM, n), lambda i: (i, 0)),
            pl.BlockSpec((BM, 128), lambda i: (i, 0)),
            pl.BlockSpec((BM, 128), lambda i: (i, 0)),
            pl.BlockSpec((BM, H), lambda i: (i, 0)),
        ],
        out_shape=[
            jax.ShapeDtypeStruct((m, n), jnp.float32),
            jax.ShapeDtypeStruct((m, 128), jnp.float32),
            jax.ShapeDtypeStruct((m, 128), jnp.float32),
            jax.ShapeDtypeStruct((m, H), jnp.float32),
        ],
    )(x, W, asv, adv)


def _mm_att(acc, den, wsp, hp, b, W, asv, adv, Hp, H):
    m, din = acc.shape
    n = W.shape[1]
    ch = asv.shape[1]
    return pl.pallas_call(
        functools.partial(_mm_att_body, Hp=Hp, H=H),
        grid=(m // BM,),
        in_specs=[
            pl.BlockSpec((BM, din), lambda i: (i, 0)),
            pl.BlockSpec((BM, 16), lambda i: (i, 0)),
            pl.BlockSpec((BM, Hp), lambda i: (i, 0)),
            pl.BlockSpec((BM, din), lambda i: (i, 0)),
            pl.BlockSpec((1, din), lambda i: (0, 0)),
            pl.BlockSpec((din, n), lambda i: (0, 0)),
            pl.BlockSpec((H, ch), lambda i: (0, 0)),
            pl.BlockSpec((H, ch), lambda i: (0, 0)),
        ],
        out_specs=[
            pl.BlockSpec((BM, n), lambda i: (i, 0)),
            pl.BlockSpec((BM, 128), lambda i: (i, 0)),
            pl.BlockSpec((BM, 128), lambda i: (i, 0)),
            pl.BlockSpec((BM, H), lambda i: (i, 0)),
        ],
        out_shape=[
            jax.ShapeDtypeStruct((m, n), jnp.float32),
            jax.ShapeDtypeStruct((m, 128), jnp.float32),
            jax.ShapeDtypeStruct((m, 128), jnp.float32),
            jax.ShapeDtypeStruct((m, H), jnp.float32),
        ],
    )(acc, den, wsp, hp, b.reshape(1, din), W, asv, adv)


def _final(acc, den, wsp, hp, b):
    m, din = acc.shape
    return pl.pallas_call(
        _final_body,
        grid=(m // BM,),
        in_specs=[
            pl.BlockSpec((BM, din), lambda i: (i, 0)),
            pl.BlockSpec((BM, 16), lambda i: (i, 0)),
            pl.BlockSpec((BM, 6), lambda i: (i, 0)),
            pl.BlockSpec((BM, din), lambda i: (i, 0)),
            pl.BlockSpec((1, din), lambda i: (0, 0)),
        ],
        out_specs=pl.BlockSpec((BM, 121), lambda i: (i, 0)),
        out_shape=jax.ShapeDtypeStruct((m, 121), jnp.float32),
    )(acc, den, wsp, hp, b.reshape(1, din))


def _make_sc_edge(D, H):
    ch = D // H
    nv = ch // 16
    mesh = plsc.VectorSubcoreMesh(core_axis_name="c", subcore_axis_name="s")

    @functools.partial(
        pl.kernel,
        out_type=[
            jax.ShapeDtypeStruct((NPAD, D), jnp.float32),
            jax.ShapeDtypeStruct((NPAD, 16), jnp.float32),
        ],
        mesh=mesh,
        compiler_params=pltpu.CompilerParams(needs_layout_passes=False),
        scratch_types=[
            pltpu.VMEM((EBLEN,), jnp.int32),
            pltpu.VMEM((2 * BS,), jnp.int32),
            pltpu.VMEM((2 * BS,), jnp.int32),
            pltpu.VMEM((BS, 128), jnp.float32),
            pltpu.VMEM((BS, 128), jnp.float32),
            pltpu.VMEM((BS, 128), jnp.float32),
            pltpu.VMEM((BS, 128), jnp.float32),
            pltpu.VMEM((BS, D), jnp.float32),
            pltpu.VMEM((BS, D), jnp.float32),
            pltpu.VMEM((SLOT_N + 1, D), jnp.float32),
            pltpu.VMEM((SLOT_N + 1, 16), jnp.float32),
            pltpu.SemaphoreType.DMA,
            pltpu.SemaphoreType.DMA,
            pltpu.SemaphoreType.DMA,
            pltpu.SemaphoreType.DMA,
            pltpu.SemaphoreType.DMA,
            pltpu.SemaphoreType.DMA,
        ],
    )
    def sc_edge(sdp_hbm, eb_hbm, as_hbm, ad_hbm, h_hbm,
                out_hbm, den_hbm,
                ebuf, sdA, sdB, arA, arB, brA, brB, hrA, hrB, acc, dacc,
                s1A, s2A, s3A, s1B, s2B, s3B):
        cid = lax.axis_index("c")
        sid = lax.axis_index("s")
        wid = sid * 2 + cid
        lane = jnp.arange(16, dtype=jnp.int32)
        z16 = jnp.zeros((16,), jnp.float32)
        mh = lane < H
        bufs = ((sdA, arA, brA, hrA, s1A, s2A, s3A),
                (sdB, arB, brB, hrB, s1B, s2B, s3B))

        pltpu.sync_copy(eb_hbm.at[pl.ds(0, EBLEN)], ebuf)

        def issue(bi, bset):
            sd, ar, br, hr, t1, t2, t3 = bset
            bic = jnp.minimum(bi, NB - 1)
            pltpu.sync_copy(sdp_hbm.at[pl.ds(bic * (2 * BS), 2 * BS)], sd)
            pltpu.async_copy(as_hbm.at[sd.at[pl.ds(0, BS)]], ar, t1)
            pltpu.async_copy(ad_hbm.at[sd.at[pl.ds(BS, BS)]], br, t2)
            pltpu.async_copy(h_hbm.at[sd.at[pl.ds(0, BS)]], hr, t3)

        def wait3(cs):
            cs[0].wait()
            cs[1].wait()
            cs[2].wait()

        def make_process(e_lo, e_hi, n0):
            def process(bi, bset):
                sd, ar, br, hr, t1, t2, t3 = bset

                def edge_body(i, _):
                    dls, wvs = [], []
                    for eo in range(4):
                        e = i * 4 + eo
                        ge = bi * BS + e
                        dstv = plsc.load_gather(
                            sd, [jnp.full((16,), BS + e, jnp.int32)])[0]
                        valid = (ge >= e_lo) & (ge < e_hi)
                        dl = jnp.where(valid, dstv - n0, SLOT_N)
                        a16 = ar[e, pl.ds(0, 16)]
                        b16 = br[e, pl.ds(0, 16)]
                        al = a16 + b16
                        al = jnp.where(al > 0, al, 0.2 * al)
                        wv = jnp.exp(al)
                        wv0 = jnp.where(mh & jnp.full((16,), valid), wv, 0.0)
                        plsc.addupdate(dacc.at[dl, :], wv0)
                        dls.append(dl)
                        wvs.append(wv)
                    for eo in range(4):
                        e = i * 4 + eo
                        dl, wv = dls[eo], wvs[eo]
                        for h in range(H):
                            w_sc = wv[h]
                            for v in range(nv):
                                cb = h * ch + v * 16
                                plsc.addupdate(acc.at[dl, pl.ds(cb, 16)],
                                               hr[e, pl.ds(cb, 16)] * w_sc)
                    return 0

                lax.fori_loop(0, BS // 4, edge_body, 0)
            return process

        def pass_body(p, _):
            slot = p * NTILE + wid
            n0 = slot * SLOT_N

            def zero_row(r, _):
                for v in range(nv * H):
                    acc[r, pl.ds(v * 16, 16)] = z16
                dacc[r, :] = z16
                return 0

            lax.fori_loop(0, SLOT_N, zero_row, 0)

            e_lo = plsc.load_gather(ebuf, [jnp.full((16,), slot, jnp.int32)])[0]
            e_hi = plsc.load_gather(
                ebuf, [jnp.full((16,), slot + 1, jnp.int32)])[0]
            b_lo = e_lo // BS
            b_hi = (e_hi + BS - 1) // BS
            nit = (b_hi - b_lo + 1) // 2
            process = make_process(e_lo, e_hi, n0)

            issue(b_lo, bufs[0])

            def pair(i, _):
                b0 = b_lo + 2 * i
                wait3((pltpu.make_async_copy(
                    as_hbm.at[bufs[0][0].at[pl.ds(0, BS)]], bufs[0][1],
                    bufs[0][4]),
                       pltpu.make_async_copy(
                    ad_hbm.at[bufs[0][0].at[pl.ds(BS, BS)]], bufs[0][2],
                    bufs[0][5]),
                       pltpu.make_async_copy(
                    h_hbm.at[bufs[0][0].at[pl.ds(0, BS)]], bufs[0][3],
                    bufs[0][6])))
                issue(b0 + 1, bufs[1])
                process(b0, bufs[0])
                wait3((pltpu.make_async_copy(
                    as_hbm.at[bufs[1][0].at[pl.ds(0, BS)]], bufs[1][1],
                    bufs[1][4]),
                       pltpu.make_async_copy(
                    ad_hbm.at[bufs[1][0].at[pl.ds(BS, BS)]], bufs[1][2],
                    bufs[1][5]),
                       pltpu.make_async_copy(
                    h_hbm.at[bufs[1][0].at[pl.ds(0, BS)]], bufs[1][3],
                    bufs[1][6])))
                issue(b0 + 2, bufs[0])
                process(b0 + 1, bufs[1])
                return 0

            lax.fori_loop(0, nit, pair, 0)
            wait3((pltpu.make_async_copy(
                as_hbm.at[bufs[0][0].at[pl.ds(0, BS)]], bufs[0][1],
                bufs[0][4]),
                   pltpu.make_async_copy(
                ad_hbm.at[bufs[0][0].at[pl.ds(BS, BS)]], bufs[0][2],
                bufs[0][5]),
                   pltpu.make_async_copy(
                h_hbm.at[bufs[0][0].at[pl.ds(0, BS)]], bufs[0][3],
                bufs[0][6])))
            pltpu.sync_copy(acc.at[pl.ds(0, SLOT_N)],
                            out_hbm.at[pl.ds(n0, SLOT_N)])
            pltpu.sync_copy(dacc.at[pl.ds(0, SLOT_N)],
                            den_hbm.at[pl.ds(n0, SLOT_N)])
            return 0

        lax.fori_loop(0, PASSES, pass_body, 0)

    return sc_edge


_sc_edge_l12 = _make_sc_edge(1024, 4)
_sc_edge_l3 = _make_sc_edge(768, 6)


def kernel(x, edge_index, W1, as1, ad1, b1, W2, as2, ad2, b2, W3, as3, ad3, b3):
    order = jnp.argsort(edge_index[1])
    ssrc = edge_index[0][order]
    sdst = edge_index[1][order]
    sdp = jnp.concatenate(
        [ssrc.reshape(NB, BS), sdst.reshape(NB, BS)], axis=1).reshape(-1)
    eb = jnp.searchsorted(
        sdst, jnp.arange(NSLOT + 1, dtype=jnp.int32) * SLOT_N).astype(jnp.int32)
    eb = jnp.concatenate(
        [eb, jnp.full((EBLEN - NSLOT - 1,), EE, jnp.int32)])

    W3p = jnp.pad(W3.reshape(1024, 6, 121),
                  ((0, 0), (0, 0), (0, 7))).reshape(1024, 768)
    as3p = jnp.pad(as3, ((0, 0), (0, 7)))
    ad3p = jnp.pad(ad3, ((0, 0), (0, 7)))
    b3p = jnp.pad(b3.reshape(6, 121), ((0, 0), (0, 7))).reshape(768)

    h1, as1n, ad1n, ws1 = _mm_att1(x, W1, as1, ad1, 4)
    acc1, den1 = _sc_edge_l12(sdp, eb, as1n, ad1n, h1)
    h2, as2n, ad2n, ws2 = _mm_att(acc1[:NN], den1[:NN], ws1, h1, b1,
                                  W2, as2, ad2, 4, 4)
    acc2, den2 = _sc_edge_l12(sdp, eb, as2n, ad2n, h2)
    h3, as3n, ad3n, ws3 = _mm_att(acc2[:NN], den2[:NN], ws2, h2, b2,
                                  W3p, as3p, ad3p, 4, 6)
    acc3, den3 = _sc_edge_l3(sdp, eb, as3n, ad3n, h3)
    return _final(acc3[:NN], den3[:NN], ws3, h3, b3p)


# speedup vs baseline: 2.2693x; 2.2693x over previous
import functools

import jax
import jax.numpy as jnp
from jax import lax
from jax.experimental import pallas as pl
from jax.experimental.pallas import tpu as pltpu
from jax.experimental.pallas import tpu_sc as plsc

NN = 10000
EE = 320000
SLOT_N = 32
NSLOT = 320
NPAD = NSLOT * SLOT_N
NTILE = 32
PASSES = NSLOT // NTILE
EBLEN = 328
BS = 32
NB = EE // BS
BM = 400


def _att_epilogue(h, asv, adv, as_ref, ad_ref, ws_ref, H):
    bm = h.shape[0]
    ch = asv.shape[1]
    h3 = h.reshape(bm, H, ch)
    a_s = (h3 * asv[None, :, :]).sum(-1)
    a_d = (h3 * adv[None, :, :]).sum(-1)
    pad = jnp.zeros((bm, 128 - H), jnp.float32)
    as_ref[...] = jnp.concatenate([a_s, pad], axis=1)
    ad_ref[...] = jnp.concatenate([a_d, pad], axis=1)
    al = a_s + a_d
    al = jnp.where(al > 0, al, 0.2 * al)
    ws_ref[...] = jnp.exp(al)


def _mm_att1_body(x_ref, w_ref, asv_ref, adv_ref, h_ref, as_ref, ad_ref,
                  ws_ref, *, H):
    h = jnp.dot(x_ref[...], w_ref[...], preferred_element_type=jnp.float32)
    h_ref[...] = h
    _att_epilogue(h, asv_ref[...], adv_ref[...], as_ref, ad_ref, ws_ref, H)


def _mm_att_body(acc_ref, den_ref, wsp_ref, hp_ref, b_ref, w_ref, asv_ref,
                 adv_ref, h_ref, as_ref, ad_ref, ws_ref, *, Hp, H):
    bm = acc_ref.shape[0]
    din = acc_ref.shape[1]
    chp = din // Hp
    acc = acc_ref[...].reshape(bm, Hp, chp)
    hp = hp_ref[...].reshape(bm, Hp, chp)
    wsp = wsp_ref[...][:, :, None]
    den = den_ref[...][:, :Hp][:, :, None]
    xn = (acc + wsp * hp) / (den + wsp + 1e-16)
    xn = xn.reshape(bm, din) + b_ref[...]
    xn = jnp.where(xn > 0, xn, jnp.exp(xn) - 1.0)
    h = jnp.dot(xn, w_ref[...], preferred_element_type=jnp.float32)
    h_ref[...] = h
    _att_epilogue(h, asv_ref[...], adv_ref[...], as_ref, ad_ref, ws_ref, H)


def _final_body(acc_ref, den_ref, wsp_ref, hp_ref, b_ref, o_ref):
    Hp, chp = 6, 128
    bm = acc_ref.shape[0]
    acc = acc_ref[...].reshape(bm, Hp, chp)
    hp = hp_ref[...].reshape(bm, Hp, chp)
    wsp = wsp_ref[...][:, :, None]
    den = den_ref[...][:, :Hp][:, :, None]
    y = (acc + wsp * hp) / (den + wsp + 1e-16)
    y = y + b_ref[...].reshape(1, Hp, chp)
    o_ref[...] = y.mean(axis=1)[:, :121]


def _mm_att1(x, W, asv, adv, H):
    m, k = x.shape
    n = W.shape[1]
    ch = asv.shape[1]
    return pl.pallas_call(
        functools.partial(_mm_att1_body, H=H),
        grid=(m // BM,),
        in_specs=[
            pl.BlockSpec((BM, k), lambda i: (i, 0)),
            pl.BlockSpec((k, n), lambda i: (0, 0)),
            pl.BlockSpec((H, ch), lambda i: (0, 0)),
            pl.BlockSpec((H, ch), lambda i: (0, 0)),
        ],
        out_specs=[
            pl.BlockSpec((BM, n), lambda i: (i, 0)),
            pl.BlockSpec((BM, 128), lambda i: (i, 0)),
            pl.BlockSpec((BM, 128), lambda i: (i, 0)),
            pl.BlockSpec((BM, H), lambda i: (i, 0)),
        ],
        out_shape=[
            jax.ShapeDtypeStruct((m, n), jnp.float32),
            jax.ShapeDtypeStruct((m, 128), jnp.float32),
            jax.ShapeDtypeStruct((m, 128), jnp.float32),
            jax.ShapeDtypeStruct((m, H), jnp.float32),
        ],
    )(x, W, asv, adv)


def _mm_att(acc, den, wsp, hp, b, W, asv, adv, Hp, H):
    m, din = acc.shape
    n = W.shape[1]
    ch = asv.shape[1]
    return pl.pallas_call(
        functools.partial(_mm_att_body, Hp=Hp, H=H),
        grid=(m // BM,),
        in_specs=[
            pl.BlockSpec((BM, din), lambda i: (i, 0)),
            pl.BlockSpec((BM, 16), lambda i: (i, 0)),
            pl.BlockSpec((BM, Hp), lambda i: (i, 0)),
            pl.BlockSpec((BM, din), lambda i: (i, 0)),
            pl.BlockSpec((1, din), lambda i: (0, 0)),
            pl.BlockSpec((din, n), lambda i: (0, 0)),
            pl.BlockSpec((H, ch), lambda i: (0, 0)),
            pl.BlockSpec((H, ch), lambda i: (0, 0)),
        ],
        out_specs=[
            pl.BlockSpec((BM, n), lambda i: (i, 0)),
            pl.BlockSpec((BM, 128), lambda i: (i, 0)),
            pl.BlockSpec((BM, 128), lambda i: (i, 0)),
            pl.BlockSpec((BM, H), lambda i: (i, 0)),
        ],
        out_shape=[
            jax.ShapeDtypeStruct((m, n), jnp.float32),
            jax.ShapeDtypeStruct((m, 128), jnp.float32),
            jax.ShapeDtypeStruct((m, 128), jnp.float32),
            jax.ShapeDtypeStruct((m, H), jnp.float32),
        ],
    )(acc, den, wsp, hp, b.reshape(1, din), W, asv, adv)


def _final(acc, den, wsp, hp, b):
    m, din = acc.shape
    return pl.pallas_call(
        _final_body,
        grid=(m // BM,),
        in_specs=[
            pl.BlockSpec((BM, din), lambda i: (i, 0)),
            pl.BlockSpec((BM, 16), lambda i: (i, 0)),
            pl.BlockSpec((BM, 6), lambda i: (i, 0)),
            pl.BlockSpec((BM, din), lambda i: (i, 0)),
            pl.BlockSpec((1, din), lambda i: (0, 0)),
        ],
        out_specs=pl.BlockSpec((BM, 121), lambda i: (i, 0)),
        out_shape=jax.ShapeDtypeStruct((m, 121), jnp.float32),
    )(acc, den, wsp, hp, b.reshape(1, din))


def _make_sc_edge(D, H):
    ch = D // H
    nv = ch // 16
    mesh = plsc.VectorSubcoreMesh(core_axis_name="c", subcore_axis_name="s")

    @functools.partial(
        pl.kernel,
        out_type=[
            jax.ShapeDtypeStruct((NPAD, D), jnp.float32),
            jax.ShapeDtypeStruct((NPAD, 16), jnp.float32),
        ],
        mesh=mesh,
        compiler_params=pltpu.CompilerParams(needs_layout_passes=False),
        scratch_types=[
            pltpu.VMEM((EBLEN,), jnp.int32),
            pltpu.VMEM((2 * BS,), jnp.int32),
            pltpu.VMEM((2 * BS,), jnp.int32),
            pltpu.VMEM((BS, 128), jnp.float32),
            pltpu.VMEM((BS, 128), jnp.float32),
            pltpu.VMEM((BS, 128), jnp.float32),
            pltpu.VMEM((BS, 128), jnp.float32),
            pltpu.VMEM((BS, D), jnp.float32),
            pltpu.VMEM((BS, D), jnp.float32),
            pltpu.VMEM((SLOT_N + 1, D), jnp.float32),
            pltpu.VMEM((SLOT_N + 1, 16), jnp.float32),
            pltpu.SemaphoreType.DMA,
            pltpu.SemaphoreType.DMA,
            pltpu.SemaphoreType.DMA,
            pltpu.SemaphoreType.DMA,
            pltpu.SemaphoreType.DMA,
            pltpu.SemaphoreType.DMA,
        ],
    )
    def sc_edge(sdp_hbm, eb_hbm, as_hbm, ad_hbm, h_hbm,
                out_hbm, den_hbm,
                ebuf, sdA, sdB, arA, arB, brA, brB, hrA, hrB, acc, dacc,
                s1A, s2A, s3A, s1B, s2B, s3B):
        cid = lax.axis_index("c")
        sid = lax.axis_index("s")
        wid = sid * 2 + cid
        lane = jnp.arange(16, dtype=jnp.int32)
        z16 = jnp.zeros((16,), jnp.float32)
        mh = lane < H
        bufs = ((sdA, arA, brA, hrA, s1A, s2A, s3A),
                (sdB, arB, brB, hrB, s1B, s2B, s3B))

        pltpu.sync_copy(eb_hbm.at[pl.ds(0, EBLEN)], ebuf)

        def issue(bi, bset):
            sd, ar, br, hr, t1, t2, t3 = bset
            bic = jnp.minimum(bi, NB - 1)
            pltpu.sync_copy(sdp_hbm.at[pl.ds(bic * (2 * BS), 2 * BS)], sd)
            pltpu.async_copy(as_hbm.at[sd.at[pl.ds(0, BS)]], ar, t1)
            pltpu.async_copy(ad_hbm.at[sd.at[pl.ds(BS, BS)]], br, t2)
            pltpu.async_copy(h_hbm.at[sd.at[pl.ds(0, BS)]], hr, t3)

        def wait3(cs):
            cs[0].wait()
            cs[1].wait()
            cs[2].wait()

        def make_process(e_lo, e_hi, n0):
            def process(bi, bset):
                sd, ar, br, hr, t1, t2, t3 = bset

                @functools.partial(plsc.parallel_loop, 0, BS, unroll=2)
                def edge_body(e):
                    ge = bi * BS + e
                    dstv = plsc.load_gather(
                        sd, [jnp.full((16,), BS + e, jnp.int32)])[0]
                    valid = (ge >= e_lo) & (ge < e_hi)
                    dl = jnp.where(valid, dstv - n0, SLOT_N)
                    a16 = ar[e, pl.ds(0, 16)]
                    b16 = br[e, pl.ds(0, 16)]
                    al = a16 + b16
                    al = jnp.where(al > 0, al, 0.2 * al)
                    wv = jnp.exp(al)
                    wv0 = jnp.where(mh & jnp.full((16,), valid), wv, 0.0)
                    plsc.addupdate(dacc.at[dl, :], wv0)
                    for h in range(H):
                        w_sc = wv[h]
                        for v in range(nv):
                            cb = h * ch + v * 16
                            plsc.addupdate(acc.at[dl, pl.ds(cb, 16)],
                                           hr[e, pl.ds(cb, 16)] * w_sc)
            return process

        def pass_body(p, _):
            slot = p * NTILE + wid
            n0 = slot * SLOT_N

            def zero_row(r, _):
                for v in range(nv * H):
                    acc[r, pl.ds(v * 16, 16)] = z16
                dacc[r, :] = z16
                return 0

            lax.fori_loop(0, SLOT_N, zero_row, 0)

            e_lo = plsc.load_gather(ebuf, [jnp.full((16,), slot, jnp.int32)])[0]
            e_hi = plsc.load_gather(
                ebuf, [jnp.full((16,), slot + 1, jnp.int32)])[0]
            b_lo = e_lo // BS
            b_hi = (e_hi + BS - 1) // BS
            nit = (b_hi - b_lo + 1) // 2
            process = make_process(e_lo, e_hi, n0)

            issue(b_lo, bufs[0])

            def pair(i, _):
                b0 = b_lo + 2 * i
                wait3((pltpu.make_async_copy(
                    as_hbm.at[bufs[0][0].at[pl.ds(0, BS)]], bufs[0][1],
                    bufs[0][4]),
                       pltpu.make_async_copy(
                    ad_hbm.at[bufs[0][0].at[pl.ds(BS, BS)]], bufs[0][2],
                    bufs[0][5]),
                       pltpu.make_async_copy(
                    h_hbm.at[bufs[0][0].at[pl.ds(0, BS)]], bufs[0][3],
                    bufs[0][6])))
                issue(b0 + 1, bufs[1])
                process(b0, bufs[0])
                wait3((pltpu.make_async_copy(
                    as_hbm.at[bufs[1][0].at[pl.ds(0, BS)]], bufs[1][1],
                    bufs[1][4]),
                       pltpu.make_async_copy(
                    ad_hbm.at[bufs[1][0].at[pl.ds(BS, BS)]], bufs[1][2],
                    bufs[1][5]),
                       pltpu.make_async_copy(
                    h_hbm.at[bufs[1][0].at[pl.ds(0, BS)]], bufs[1][3],
                    bufs[1][6])))
                issue(b0 + 2, bufs[0])
                process(b0 + 1, bufs[1])
                return 0

            lax.fori_loop(0, nit, pair, 0)
            wait3((pltpu.make_async_copy(
                as_hbm.at[bufs[0][0].at[pl.ds(0, BS)]], bufs[0][1],
                bufs[0][4]),
                   pltpu.make_async_copy(
                ad_hbm.at[bufs[0][0].at[pl.ds(BS, BS)]], bufs[0][2],
                bufs[0][5]),
                   pltpu.make_async_copy(
                h_hbm.at[bufs[0][0].at[pl.ds(0, BS)]], bufs[0][3],
                bufs[0][6])))
            pltpu.sync_copy(acc.at[pl.ds(0, SLOT_N)],
                            out_hbm.at[pl.ds(n0, SLOT_N)])
            pltpu.sync_copy(dacc.at[pl.ds(0, SLOT_N)],
                            den_hbm.at[pl.ds(n0, SLOT_N)])
            return 0

        lax.fori_loop(0, PASSES, pass_body, 0)

    return sc_edge


_sc_edge_l12 = _make_sc_edge(1024, 4)
_sc_edge_l3 = _make_sc_edge(768, 6)


def kernel(x, edge_index, W1, as1, ad1, b1, W2, as2, ad2, b2, W3, as3, ad3, b3):
    order = jnp.argsort(edge_index[1])
    ssrc = edge_index[0][order]
    sdst = edge_index[1][order]
    sdp = jnp.concatenate(
        [ssrc.reshape(NB, BS), sdst.reshape(NB, BS)], axis=1).reshape(-1)
    eb = jnp.searchsorted(
        sdst, jnp.arange(NSLOT + 1, dtype=jnp.int32) * SLOT_N).astype(jnp.int32)
    eb = jnp.concatenate(
        [eb, jnp.full((EBLEN - NSLOT - 1,), EE, jnp.int32)])

    W3p = jnp.pad(W3.reshape(1024, 6, 121),
                  ((0, 0), (0, 0), (0, 7))).reshape(1024, 768)
    as3p = jnp.pad(as3, ((0, 0), (0, 7)))
    ad3p = jnp.pad(ad3, ((0, 0), (0, 7)))
    b3p = jnp.pad(b3.reshape(6, 121), ((0, 0), (0, 7))).reshape(768)

    h1, as1n, ad1n, ws1 = _mm_att1(x, W1, as1, ad1, 4)
    acc1, den1 = _sc_edge_l12(sdp, eb, as1n, ad1n, h1)
    h2, as2n, ad2n, ws2 = _mm_att(acc1[:NN], den1[:NN], ws1, h1, b1,
                                  W2, as2, ad2, 4, 4)
    acc2, den2 = _sc_edge_l12(sdp, eb, as2n, ad2n, h2)
    h3, as3n, ad3n, ws3 = _mm_att(acc2[:NN], den2[:NN], ws2, h2, b2,
                                  W3p, as3p, ad3p, 4, 6)
    acc3, den3 = _sc_edge_l3(sdp, eb, as3n, ad3n, h3)
    return _final(acc3[:NN], den3[:NN], ws3, h3, b3p)
